# baseline (speedup 1.0000x reference)
import functools

import jax
import jax.numpy as jnp
from jax import lax
from jax.experimental import pallas as pl
from jax.experimental.pallas import tpu as pltpu
from jax.experimental.pallas import tpu_sc as plsc

BATCH = 16384
HIST = 200
N = BATCH * HIST
VOCAB_N = 1_000_000
NUM_WORKERS = 32
BPW = N // NUM_WORKERS
CA = 16_640
CB = 8_960
CHUNK = CA + CB
NCHUNK = BPW // CHUNK
SEG = 10_000
NSEG = VOCAB_N // SEG


def _make_kernel():
    mesh = plsc.VectorSubcoreMesh(core_axis_name="c", subcore_axis_name="s")

    @functools.partial(
        pl.kernel,
        mesh=mesh,
        out_type=jax.ShapeDtypeStruct((N,), jnp.float32),
        scratch_types=[
            pltpu.VMEM_SHARED((VOCAB_N,), jnp.float32),
            pltpu.VMEM((CA,), jnp.int32),
            pltpu.VMEM((CB,), jnp.int32),
            pltpu.VMEM((CA,), jnp.float32),
            pltpu.VMEM((CB,), jnp.float32),
            pltpu.SemaphoreType.DMA,
            pltpu.SemaphoreType.DMA,
        ],
    )
    def gather_kernel(tok_hbm, tab_hbm, out_hbm, tab_sp, idxa, idxb,
                      vala, valb, sga, sgb):
        s = lax.axis_index("s")
        wid = s * 2 + lax.axis_index("c")
        base = wid * BPW

        for r in range((NSEG + 15) // 16):

            @pl.when(r * 16 + s < NSEG)
            def _():
                toff = (r * 16 + s) * SEG
                pltpu.sync_copy(tab_hbm.at[pl.ds(toff, SEG)],
                                vala.at[pl.ds(0, SEG)])
                pltpu.sync_copy(vala.at[pl.ds(0, SEG)],
                                tab_sp.at[pl.ds(toff, SEG)])

        plsc.subcore_barrier()

        for i in range(NCHUNK):
            off_a = base + i * CHUNK
            off_b = off_a + CA
            pltpu.async_copy(tok_hbm.at[pl.ds(off_a, CA)], idxa, sga)
            pltpu.async_copy(tok_hbm.at[pl.ds(off_b, CB)], idxb, sgb)
            pltpu.make_async_copy(tok_hbm.at[pl.ds(off_a, CA)], idxa,
                                  sga).wait()
            pltpu.make_async_copy(tok_hbm.at[pl.ds(off_b, CB)], idxb,
                                  sgb).wait()
            pltpu.async_copy(tab_sp.at[idxa], vala, sga)
            pltpu.async_copy(tab_hbm.at[idxb], valb, sgb)
            pltpu.make_async_copy(tab_sp.at[idxa], vala, sga).wait()
            pltpu.make_async_copy(tab_hbm.at[idxb], valb, sgb).wait()
            pltpu.async_copy(vala, out_hbm.at[pl.ds(off_a, CA)], sga)
            pltpu.async_copy(valb, out_hbm.at[pl.ds(off_b, CB)], sgb)
            pltpu.make_async_copy(vala, out_hbm.at[pl.ds(off_a, CA)],
                                  sga).wait()
            pltpu.make_async_copy(valb, out_hbm.at[pl.ds(off_b, CB)],
                                  sgb).wait()

    return gather_kernel


_GATHER = _make_kernel()


def kernel(tokens, vocab_table):
    flat = tokens.reshape(N)
    out = _GATHER(flat, vocab_table)
    return out.reshape(BATCH, HIST)


# speedup vs baseline: 1.1602x; 1.1602x over previous
import functools

import jax
import jax.numpy as jnp
from jax import lax
from jax.experimental import pallas as pl
from jax.experimental.pallas import tpu as pltpu
from jax.experimental.pallas import tpu_sc as plsc

BATCH = 16384
HIST = 200
N = BATCH * HIST
VOCAB_N = 1_000_000
NUM_WORKERS = 32
BPW = N // NUM_WORKERS
CHUNK = 12_800
HALF = CHUNK // 2
NCHUNK = BPW // CHUNK
SEG = 10_000
NSEG = VOCAB_N // SEG


def _make_kernel():
    mesh = plsc.VectorSubcoreMesh(core_axis_name="c", subcore_axis_name="s")

    @functools.partial(
        pl.kernel,
        mesh=mesh,
        out_type=jax.ShapeDtypeStruct((N,), jnp.float32),
        scratch_types=[
            pltpu.VMEM_SHARED((VOCAB_N,), jnp.float32),
            pltpu.VMEM((CHUNK,), jnp.int32),
            pltpu.VMEM((CHUNK,), jnp.int32),
            pltpu.VMEM((CHUNK,), jnp.float32),
            pltpu.VMEM((CHUNK,), jnp.float32),
            pltpu.SemaphoreType.DMA,
            pltpu.SemaphoreType.DMA,
            pltpu.SemaphoreType.DMA,
            pltpu.SemaphoreType.DMA,
            pltpu.SemaphoreType.DMA,
            pltpu.SemaphoreType.DMA,
        ],
    )
    def gather_kernel(tok_hbm, tab_hbm, out_hbm, tab_sp, idx0, idx1,
                      val0, val1, si0, si1, so0, so1, sga, sgb):
        s = lax.axis_index("s")
        wid = s * 2 + lax.axis_index("c")
        base = wid * BPW
        idx = (idx0, idx1)
        val = (val0, val1)
        sem_i = (si0, si1)
        sem_o = (so0, so1)

        for r in range((NSEG + 15) // 16):

            @pl.when(r * 16 + s < NSEG)
            def _():
                toff = (r * 16 + s) * SEG
                pltpu.sync_copy(tab_hbm.at[pl.ds(toff, SEG)],
                                val0.at[pl.ds(0, SEG)])
                pltpu.sync_copy(val0.at[pl.ds(0, SEG)],
                                tab_sp.at[pl.ds(toff, SEG)])

        plsc.subcore_barrier()

        pltpu.async_copy(tok_hbm.at[pl.ds(base, CHUNK)], idx0, si0)
        for i in range(NCHUNK):
            b = i % 2
            if i + 1 < NCHUNK:
                pltpu.async_copy(
                    tok_hbm.at[pl.ds(base + (i + 1) * CHUNK, CHUNK)],
                    idx[1 - b], sem_i[1 - b])
            if i >= 2:
                pltpu.make_async_copy(
                    val[b], out_hbm.at[pl.ds(base + (i - 2) * CHUNK, CHUNK)],
                    sem_o[b]).wait()
            pltpu.make_async_copy(
                tok_hbm.at[pl.ds(base + i * CHUNK, CHUNK)], idx[b],
                sem_i[b]).wait()
            pltpu.async_copy(tab_sp.at[idx[b].at[pl.ds(0, HALF)]],
                             val[b].at[pl.ds(0, HALF)], sga)
            pltpu.async_copy(tab_sp.at[idx[b].at[pl.ds(HALF, HALF)]],
                             val[b].at[pl.ds(HALF, HALF)], sgb)
            pltpu.make_async_copy(tab_sp.at[idx[b].at[pl.ds(0, HALF)]],
                                  val[b].at[pl.ds(0, HALF)], sga).wait()
            pltpu.make_async_copy(tab_sp.at[idx[b].at[pl.ds(HALF, HALF)]],
                                  val[b].at[pl.ds(HALF, HALF)], sgb).wait()
            pltpu.async_copy(
                val[b], out_hbm.at[pl.ds(base + i * CHUNK, CHUNK)], sem_o[b])
        for i in range(NCHUNK - 2, NCHUNK):
            b = i % 2
            pltpu.make_async_copy(
                val[b], out_hbm.at[pl.ds(base + i * CHUNK, CHUNK)],
                sem_o[b]).wait()

    return gather_kernel


_GATHER = _make_kernel()


def kernel(tokens, vocab_table):
    flat = tokens.reshape(N)
    out = _GATHER(flat, vocab_table)
    return out.reshape(BATCH, HIST)


# speedup vs baseline: 1.1711x; 1.0093x over previous
import functools

import jax
import jax.numpy as jnp
from jax import lax
from jax.experimental import pallas as pl
from jax.experimental.pallas import tpu as pltpu
from jax.experimental.pallas import tpu_sc as plsc

BATCH = 16384
HIST = 200
N = BATCH * HIST
VOCAB_N = 1_000_000
NUM_WORKERS = 32
BPW = N // NUM_WORKERS
CHUNK = 12_800
HALF = CHUNK // 2
NCHUNK = BPW // CHUNK
SEG = 10_000
NSEG = VOCAB_N // SEG


def _make_kernel():
    mesh = plsc.VectorSubcoreMesh(core_axis_name="c", subcore_axis_name="s")

    @functools.partial(
        pl.kernel,
        mesh=mesh,
        out_type=jax.ShapeDtypeStruct((N,), jnp.float32),
        scratch_types=[
            pltpu.VMEM_SHARED((VOCAB_N,), jnp.float32),
            pltpu.VMEM((CHUNK,), jnp.int32),
            pltpu.VMEM((CHUNK,), jnp.int32),
            pltpu.VMEM((CHUNK,), jnp.float32),
            pltpu.VMEM((CHUNK,), jnp.float32),
            pltpu.SemaphoreType.DMA,
            pltpu.SemaphoreType.DMA,
            pltpu.SemaphoreType.DMA,
            pltpu.SemaphoreType.DMA,
            pltpu.SemaphoreType.DMA,
            pltpu.SemaphoreType.DMA,
        ],
    )
    def gather_kernel(tok_hbm, tab_hbm, out_hbm, tab_sp, idx0, idx1,
                      val0, val1, si0, si1, so0, so1, sga, sgb):
        s = lax.axis_index("s")
        wid = s * 2 + lax.axis_index("c")
        base = wid * BPW
        idx = (idx0, idx1)
        val = (val0, val1)
        sem_i = (si0, si1)
        sem_o = (so0, so1)

        pltpu.async_copy(tok_hbm.at[pl.ds(base, CHUNK)], idx0, si0)

        for r in range((NSEG + 15) // 16):

            @pl.when(r * 16 + s < NSEG)
            def _():
                toff = (r * 16 + s) * SEG
                pltpu.sync_copy(tab_hbm.at[pl.ds(toff, SEG)],
                                val0.at[pl.ds(0, SEG)])
                pltpu.sync_copy(val0.at[pl.ds(0, SEG)],
                                tab_sp.at[pl.ds(toff, SEG)])

        plsc.subcore_barrier()
        for i in range(NCHUNK):
            b = i % 2
            if i + 1 < NCHUNK:
                pltpu.async_copy(
                    tok_hbm.at[pl.ds(base + (i + 1) * CHUNK, CHUNK)],
                    idx[1 - b], sem_i[1 - b])
            if i >= 2:
                pltpu.make_async_copy(
                    val[b], out_hbm.at[pl.ds(base + (i - 2) * CHUNK, CHUNK)],
                    sem_o[b]).wait()
            pltpu.make_async_copy(
                tok_hbm.at[pl.ds(base + i * CHUNK, CHUNK)], idx[b],
                sem_i[b]).wait()
            pltpu.async_copy(tab_sp.at[idx[b].at[pl.ds(0, HALF)]],
                             val[b].at[pl.ds(0, HALF)], sga)
            pltpu.async_copy(tab_sp.at[idx[b].at[pl.ds(HALF, HALF)]],
                             val[b].at[pl.ds(HALF, HALF)], sgb)
            pltpu.make_async_copy(tab_sp.at[idx[b].at[pl.ds(0, HALF)]],
                                  val[b].at[pl.ds(0, HALF)], sga).wait()
            pltpu.make_async_copy(tab_sp.at[idx[b].at[pl.ds(HALF, HALF)]],
                                  val[b].at[pl.ds(HALF, HALF)], sgb).wait()
            pltpu.async_copy(
                val[b], out_hbm.at[pl.ds(base + i * CHUNK, CHUNK)], sem_o[b])
        for i in range(NCHUNK - 2, NCHUNK):
            b = i % 2
            pltpu.make_async_copy(
                val[b], out_hbm.at[pl.ds(base + i * CHUNK, CHUNK)],
                sem_o[b]).wait()

    return gather_kernel


_GATHER = _make_kernel()


def kernel(tokens, vocab_table):
    flat = tokens.reshape(N)
    out = _GATHER(flat, vocab_table)
    return out.reshape(BATCH, HIST)
